# baseline (speedup 1.0000x reference)
import functools

import jax
import jax.numpy as jnp
from jax import lax
from jax.experimental import pallas as pl
from jax.experimental.pallas import tpu as pltpu
from jax.experimental.pallas import tpu_sc as plsc

_NC = 2
_NS = 16
_NW = _NC * _NS
_UNIT = 1024
_K = 2
_XCH = 400


def _round_up(a, b):
    return (a + b - 1) // b * b


@functools.lru_cache(maxsize=None)
def _sc_scatter(n_tbl, e_rows, t, n_x):
    npt = n_tbl // _NS
    q, rem = divmod(e_rows, _NW)
    mesh = plsc.VectorSubcoreMesh(core_axis_name="c", subcore_axis_name="s")

    def body(xin, ei, agg_out, deg_out, tbl,
             acc, deg_acc, src_v0, src_v1, dst_v0, dst_v1, rows_v0, rows_v1,
             xbuf, tbuf, zb1, ones_v,
             xsem, isem0, isem1, gsem0, gsem1, dsem0, dsem1, asem0, asem1):
        c = lax.axis_index("c")
        s = lax.axis_index("s")
        w = c * _NS + s
        r0 = s * npt
        srcs = (src_v0, src_v1)
        dsts = (dst_v0, dst_v1)
        rows = (rows_v0, rows_v1)
        isems = (isem0, isem1)
        gsems = (gsem0, gsem1)
        dsems = (dsem0, dsem1)
        asems = (asem0, asem1)
        iota = lax.iota(jnp.int32, 16)
        z16 = jnp.zeros((16,), jnp.float32)

        for ch in range(npt // _XCH):
            c0 = r0 + ch * _XCH

            @pl.when(c0 + _XCH <= n_x)
            def _():
                pltpu.async_copy(xin.at[:, pl.ds(c0, _XCH)],
                                 xbuf.at[:, pl.ds(ch * _XCH, _XCH)], xsem)

        for i in range(_K):
            for j in range(_UNIT // 16):
                ones_v[i, pl.ds(j * 16, 16)] = jnp.ones((16,), jnp.float32)

        @pl.loop(0, npt // 16)
        def _z1(i):
            zb1[pl.ds(i * 16, 16)] = z16

        @pl.loop(0, npt // 16)
        def _z2(i):
            r16 = i * 16 + iota
            for tt in range(t):
                plsc.store_scatter(tbuf, [r16, jnp.full((16,), tt, jnp.int32)],
                                   z16)

        pltpu.sync_copy(tbuf, acc.at[pl.ds(r0, npt)])
        pltpu.sync_copy(zb1, deg_acc.at[pl.ds(r0, npt)])

        for ch in range(npt // _XCH):
            c0 = r0 + ch * _XCH

            @pl.when(c0 + _XCH <= n_x)
            def _():
                pltpu.make_async_copy(xin.at[:, pl.ds(c0, _XCH)],
                                      xbuf.at[:, pl.ds(ch * _XCH, _XCH)],
                                      xsem).wait()

        @pl.loop(0, npt // 16)
        def _tr(i):
            r16 = i * 16 + iota
            for tt in range(t):
                ft = jnp.full((16,), tt, jnp.int32)
                v = plsc.load_gather(xbuf, [ft, r16])
                plsc.store_scatter(tbuf, [r16, ft], v)

        pltpu.sync_copy(tbuf, tbl.at[c, pl.ds(r0, npt)])
        plsc.subcore_barrier()

        base = w * q + jnp.minimum(w, rem)
        nrows = q + jnp.where(w < rem, 1, 0)
        nfull = (nrows // (2 * _K)) * 2
        tblc = tbl.at[c]

        def fire_idx(b, p):
            rb = base + b * _K
            pltpu.async_copy(ei.at[0, pl.ds(rb, _K)], srcs[p], isems[p])
            pltpu.async_copy(ei.at[1, pl.ds(rb, _K)], dsts[p], isems[p])

        def wait_idx(b, p):
            rb = base + b * _K
            pltpu.make_async_copy(ei.at[0, pl.ds(rb, _K)], srcs[p],
                                  isems[p]).wait()
            pltpu.make_async_copy(ei.at[1, pl.ds(rb, _K)], dsts[p],
                                  isems[p]).wait()

        def drain_scatters(p):
            for j in range(_K):
                pltpu.make_async_copy(ones_v.at[j], deg_acc.at[srcs[p].at[j]],
                                      dsems[p]).wait()
            for j in range(_K):
                pltpu.make_async_copy(rows[p].at[j], acc.at[dsts[p].at[j]],
                                      asems[p]).wait()

        def block(b, p, drain_prev):
            wait_idx(b, p)
            gds = []
            for j in range(_K):
                gds.append(pltpu.async_copy(tblc.at[srcs[p].at[j]],
                                            rows[p].at[j], gsems[p]))
            for j in range(_K):
                pltpu.async_copy(ones_v.at[j], deg_acc.at[srcs[p].at[j]],
                                 dsems[p], add=True)
            if drain_prev:
                drain_scatters(1 - p)

            @pl.when(b + 1 < nfull)
            def _():
                fire_idx(b + 1, 1 - p)

            for d in gds:
                d.wait()
            for j in range(_K):
                pltpu.async_copy(rows[p].at[j], acc.at[dsts[p].at[j]],
                                 asems[p], add=True)

        fire_idx(0, 0)
        block(0, 0, drain_prev=False)
        block(1, 1, drain_prev=True)

        @pl.loop(1, nfull // 2)
        def _outer(gg):
            block(gg * 2, 0, drain_prev=True)
            block(gg * 2 + 1, 1, drain_prev=True)

        drain_scatters(1)

        @pl.loop(nfull * _K, nrows)
        def _tail(r):
            pltpu.sync_copy(ei.at[0, pl.ds(base + r, 1)],
                            srcs[0].at[pl.ds(0, 1)])
            pltpu.sync_copy(ei.at[1, pl.ds(base + r, 1)],
                            dsts[0].at[pl.ds(0, 1)])
            pltpu.sync_copy(tblc.at[srcs[0].at[0]], rows[0].at[0])
            pltpu.sync_copy(ones_v.at[0], deg_acc.at[srcs[0].at[0]], add=True)
            pltpu.sync_copy(rows[0].at[0], acc.at[dsts[0].at[0]], add=True)

        plsc.subcore_barrier()

        pltpu.sync_copy(deg_acc.at[pl.ds(r0, npt)],
                        deg_out.at[pl.ds(c * n_tbl + r0, npt)])
        pltpu.sync_copy(acc.at[pl.ds(r0, npt)], tbuf)

        @pl.loop(0, npt // 16)
        def _tro(i):
            r16 = i * 16 + iota
            for tt in range(t):
                v = plsc.load_gather(tbuf, [r16, jnp.full((16,), tt,
                                                          jnp.int32)])
                xbuf[tt, pl.ds(i * 16, 16)] = v

        pltpu.sync_copy(xbuf, agg_out.at[c, :, pl.ds(r0, npt)])

    return pl.kernel(
        body,
        out_type=(
            jax.ShapeDtypeStruct((_NC, t, n_tbl), jnp.float32),
            jax.ShapeDtypeStruct((_NC * n_tbl,), jnp.float32),
        ),
        mesh=mesh,
        scratch_types=(
            [pltpu.HBM((_NC, n_tbl, t), jnp.float32),
             pltpu.VMEM_SHARED((n_tbl, t), jnp.float32),
             pltpu.VMEM_SHARED((n_tbl,), jnp.float32)]
            + [pltpu.VMEM((_K, _UNIT), jnp.int32)] * 4
            + [pltpu.VMEM((_K, _UNIT, t), jnp.float32)] * 2
            + [pltpu.VMEM((t, npt), jnp.float32),
               pltpu.VMEM((npt, t), jnp.float32),
               pltpu.VMEM((npt,), jnp.float32),
               pltpu.VMEM((_K, _UNIT), jnp.float32)]
            + [pltpu.SemaphoreType.DMA] * 9
        ),
        compiler_params=pltpu.CompilerParams(use_tc_tiling_on_sc=False,
                                             needs_layout_passes=False),
    )


def _combine_body(x_ref, deg_ref, agg_ref, a1_ref, a2_ref, g_ref, b_ref,
                  wb_ref, pf_ref, out_ref):
    n = x_ref.shape[1]
    deg = jnp.maximum(deg_ref[0:1, :n] + deg_ref[1:2, :n], 1.0)
    ld = jnp.log(deg)
    dp = jax.nn.sigmoid(g_ref[0, 0])
    sw = jnp.exp(a1_ref[0, 0])
    nw = sw * jnp.tanh(a2_ref[0, 0])
    agg = agg_ref[0, :, :n] + agg_ref[1, :, :n]
    wr = x_ref[...] * jnp.exp(dp * ld)
    post = jnp.where(pf_ref[0, 0] != 0,
                     jnp.exp((dp - 1.0) * ld), jnp.ones_like(ld))
    outv = sw * wr + nw * agg * post
    out_ref[...] = outv + jnp.where(wb_ref[0, 0] != 0, b_ref[0, 0], 0.0)


def _combine(x, deg2, agg_t, a1, a2, g, b, wb, post_flag):
    t, n = x.shape
    return pl.pallas_call(
        _combine_body,
        out_shape=jax.ShapeDtypeStruct((t, n), jnp.float32),
    )(x, deg2, agg_t, a1, a2, g, b, wb, post_flag)


def _prescale_body(x_ref, deg_ref, g_ref, z_ref):
    n = x_ref.shape[1]
    dp = jax.nn.sigmoid(g_ref[0, 0])
    deg = jnp.maximum(deg_ref[0:1, :n] + deg_ref[1:2, :n], 1.0)
    f = jnp.exp((dp - 1.0) * jnp.log(deg))
    z_ref[...] = x_ref[...] * f


def _prep(x, edge_index):
    t, n = x.shape
    e = edge_index.shape[1]
    fast = (e % _UNIT == 0 and n % _XCH == 0
            and (e // _UNIT) // _NW >= 4 * _K)
    if fast:
        return x, edge_index.reshape(2, e // _UNIT, _UNIT)
    n_pad = _round_up(n, _XCH)
    e_pad = _round_up(max(e, 4 * _K * _NW * _UNIT), _NW * _UNIT)
    xin = jnp.pad(x, ((0, 0), (0, n_pad - n)))
    pad_idx = n_pad + (jnp.arange(e_pad - e, dtype=jnp.int32) % 32)
    edges = jnp.concatenate(
        [edge_index, jnp.broadcast_to(pad_idx, (2, e_pad - e))], axis=1
    ).reshape(2, e_pad // _UNIT, _UNIT)
    return xin, edges


def kernel(x, edge_index, alpha1, alpha2, gamma, bias, transpose, with_bias):
    t, n = x.shape
    xin, edges = _prep(x, edge_index)
    n_x = xin.shape[1]
    n_tbl = _round_up(n_x + 64, _NS * 3200)
    e_rows = edges.shape[1]

    scatter = _sc_scatter(n_tbl, e_rows, t, n_x)
    agg_t2, deg_flat = scatter(xin, edges)
    deg2 = deg_flat.reshape(_NC, n_tbl)
    wb = jnp.asarray(with_bias, jnp.int32).reshape(1, 1)

    def branch_plain(_):
        pf = jnp.ones((1, 1), jnp.int32)
        return _combine(x, deg2, agg_t2, alpha1, alpha2, gamma, bias, wb, pf)

    def branch_transpose(_):
        xs = pl.pallas_call(
            _prescale_body,
            out_shape=jax.ShapeDtypeStruct((t, n_x), jnp.float32),
        )(xin, deg2, gamma)
        agg_t2b, _ = scatter(xs, _prep(x, edge_index)[1])
        pf = jnp.zeros((1, 1), jnp.int32)
        return _combine(x, deg2, agg_t2b, alpha1, alpha2, gamma, bias, wb, pf)

    return lax.cond(jnp.asarray(transpose) == 0,
                    branch_plain, branch_transpose, operand=None)


# speedup vs baseline: 1.1448x; 1.1448x over previous
import functools

import jax
import jax.numpy as jnp
from jax import lax
from jax.experimental import pallas as pl
from jax.experimental.pallas import tpu as pltpu
from jax.experimental.pallas import tpu_sc as plsc

_NC = 2
_NS = 16
_NW = _NC * _NS
_UNIT = 512
_K = 4
_XCH = 400


def _round_up(a, b):
    return (a + b - 1) // b * b


@functools.lru_cache(maxsize=None)
def _sc_scatter(n_tbl, e_rows, t, n_x):
    npt = n_tbl // _NS
    q, rem = divmod(e_rows, _NW)
    mesh = plsc.VectorSubcoreMesh(core_axis_name="c", subcore_axis_name="s")

    def body(xin, ei, agg_out, deg_out, tbl,
             acc, deg_acc, src_v0, src_v1, dst_v0, dst_v1, rows_v0, rows_v1,
             xbuf, tbuf, zb1, ones_v,
             xsem, isem0, isem1, gsem0, gsem1, dsem0, dsem1, asem0, asem1):
        c = lax.axis_index("c")
        s = lax.axis_index("s")
        w = c * _NS + s
        r0 = s * npt
        srcs = (src_v0, src_v1)
        dsts = (dst_v0, dst_v1)
        rows = (rows_v0, rows_v1)
        isems = (isem0, isem1)
        gsems = (gsem0, gsem1)
        dsems = (dsem0, dsem1)
        asems = (asem0, asem1)
        iota = lax.iota(jnp.int32, 16)
        z16 = jnp.zeros((16,), jnp.float32)

        for ch in range(npt // _XCH):
            c0 = r0 + ch * _XCH

            @pl.when(c0 + _XCH <= n_x)
            def _():
                pltpu.async_copy(xin.at[:, pl.ds(c0, _XCH)],
                                 xbuf.at[:, pl.ds(ch * _XCH, _XCH)], xsem)

        for i in range(_K):
            for j in range(_UNIT // 16):
                ones_v[i, pl.ds(j * 16, 16)] = jnp.ones((16,), jnp.float32)

        @pl.loop(0, npt // 16)
        def _z1(i):
            zb1[pl.ds(i * 16, 16)] = z16

        @pl.loop(0, npt // 16)
        def _z2(i):
            r16 = i * 16 + iota
            for tt in range(t):
                plsc.store_scatter(tbuf, [r16, jnp.full((16,), tt, jnp.int32)],
                                   z16)

        pltpu.sync_copy(tbuf, acc.at[pl.ds(r0, npt)])
        pltpu.sync_copy(zb1, deg_acc.at[pl.ds(r0, npt)])

        for ch in range(npt // _XCH):
            c0 = r0 + ch * _XCH

            @pl.when(c0 + _XCH <= n_x)
            def _():
                pltpu.make_async_copy(xin.at[:, pl.ds(c0, _XCH)],
                                      xbuf.at[:, pl.ds(ch * _XCH, _XCH)],
                                      xsem).wait()

        @pl.loop(0, npt // 16)
        def _tr(i):
            r16 = i * 16 + iota
            for tt in range(t):
                ft = jnp.full((16,), tt, jnp.int32)
                v = plsc.load_gather(xbuf, [ft, r16])
                plsc.store_scatter(tbuf, [r16, ft], v)

        pltpu.sync_copy(tbuf, tbl.at[c, pl.ds(r0, npt)])
        plsc.subcore_barrier()

        base = w * q + jnp.minimum(w, rem)
        nrows = q + jnp.where(w < rem, 1, 0)
        nfull = (nrows // (2 * _K)) * 2
        tblc = tbl.at[c]

        def fire_idx(b, p):
            rb = base + b * _K
            pltpu.async_copy(ei.at[0, pl.ds(rb, _K)], srcs[p], isems[p])
            pltpu.async_copy(ei.at[1, pl.ds(rb, _K)], dsts[p], isems[p])

        def wait_idx(b, p):
            rb = base + b * _K
            pltpu.make_async_copy(ei.at[0, pl.ds(rb, _K)], srcs[p],
                                  isems[p]).wait()
            pltpu.make_async_copy(ei.at[1, pl.ds(rb, _K)], dsts[p],
                                  isems[p]).wait()

        def drain_scatters(p):
            for j in range(_K):
                pltpu.make_async_copy(ones_v.at[j], deg_acc.at[srcs[p].at[j]],
                                      dsems[p]).wait()
            for j in range(_K):
                pltpu.make_async_copy(rows[p].at[j], acc.at[dsts[p].at[j]],
                                      asems[p]).wait()

        def block(b, p, drain_prev):
            wait_idx(b, p)
            gds = []
            for j in range(_K):
                gds.append(pltpu.async_copy(tblc.at[srcs[p].at[j]],
                                            rows[p].at[j], gsems[p]))
            for j in range(_K):
                pltpu.async_copy(ones_v.at[j], deg_acc.at[srcs[p].at[j]],
                                 dsems[p], add=True)
            if drain_prev:
                drain_scatters(1 - p)

            @pl.when(b + 1 < nfull)
            def _():
                fire_idx(b + 1, 1 - p)

            for d in gds:
                d.wait()
            for j in range(_K):
                pltpu.async_copy(rows[p].at[j], acc.at[dsts[p].at[j]],
                                 asems[p], add=True)

        fire_idx(0, 0)
        block(0, 0, drain_prev=False)
        block(1, 1, drain_prev=True)

        @pl.loop(1, nfull // 2)
        def _outer(gg):
            block(gg * 2, 0, drain_prev=True)
            block(gg * 2 + 1, 1, drain_prev=True)

        drain_scatters(1)

        @pl.loop(nfull * _K, nrows)
        def _tail(r):
            pltpu.sync_copy(ei.at[0, pl.ds(base + r, 1)],
                            srcs[0].at[pl.ds(0, 1)])
            pltpu.sync_copy(ei.at[1, pl.ds(base + r, 1)],
                            dsts[0].at[pl.ds(0, 1)])
            pltpu.sync_copy(tblc.at[srcs[0].at[0]], rows[0].at[0])
            pltpu.sync_copy(ones_v.at[0], deg_acc.at[srcs[0].at[0]], add=True)
            pltpu.sync_copy(rows[0].at[0], acc.at[dsts[0].at[0]], add=True)

        plsc.subcore_barrier()

        pltpu.sync_copy(deg_acc.at[pl.ds(r0, npt)],
                        deg_out.at[pl.ds(c * n_tbl + r0, npt)])
        pltpu.sync_copy(acc.at[pl.ds(r0, npt)], tbuf)

        @pl.loop(0, npt // 16)
        def _tro(i):
            r16 = i * 16 + iota
            for tt in range(t):
                v = plsc.load_gather(tbuf, [r16, jnp.full((16,), tt,
                                                          jnp.int32)])
                xbuf[tt, pl.ds(i * 16, 16)] = v

        pltpu.sync_copy(xbuf, agg_out.at[c, :, pl.ds(r0, npt)])

    return pl.kernel(
        body,
        out_type=(
            jax.ShapeDtypeStruct((_NC, t, n_tbl), jnp.float32),
            jax.ShapeDtypeStruct((_NC * n_tbl,), jnp.float32),
        ),
        mesh=mesh,
        scratch_types=(
            [pltpu.HBM((_NC, n_tbl, t), jnp.float32),
             pltpu.VMEM_SHARED((n_tbl, t), jnp.float32),
             pltpu.VMEM_SHARED((n_tbl,), jnp.float32)]
            + [pltpu.VMEM((_K, _UNIT), jnp.int32)] * 4
            + [pltpu.VMEM((_K, _UNIT, t), jnp.float32)] * 2
            + [pltpu.VMEM((t, npt), jnp.float32),
               pltpu.VMEM((npt, t), jnp.float32),
               pltpu.VMEM((npt,), jnp.float32),
               pltpu.VMEM((_K, _UNIT), jnp.float32)]
            + [pltpu.SemaphoreType.DMA] * 9
        ),
        compiler_params=pltpu.CompilerParams(use_tc_tiling_on_sc=False,
                                             needs_layout_passes=False),
    )


def _combine_body(x_ref, deg_ref, agg_ref, a1_ref, a2_ref, g_ref, b_ref,
                  wb_ref, pf_ref, out_ref):
    n = x_ref.shape[1]
    deg = jnp.maximum(deg_ref[0:1, :n] + deg_ref[1:2, :n], 1.0)
    ld = jnp.log(deg)
    dp = jax.nn.sigmoid(g_ref[0, 0])
    sw = jnp.exp(a1_ref[0, 0])
    nw = sw * jnp.tanh(a2_ref[0, 0])
    agg = agg_ref[0, :, :n] + agg_ref[1, :, :n]
    wr = x_ref[...] * jnp.exp(dp * ld)
    post = jnp.where(pf_ref[0, 0] != 0,
                     jnp.exp((dp - 1.0) * ld), jnp.ones_like(ld))
    outv = sw * wr + nw * agg * post
    out_ref[...] = outv + jnp.where(wb_ref[0, 0] != 0, b_ref[0, 0], 0.0)


def _combine(x, deg2, agg_t, a1, a2, g, b, wb, post_flag):
    t, n = x.shape
    return pl.pallas_call(
        _combine_body,
        out_shape=jax.ShapeDtypeStruct((t, n), jnp.float32),
    )(x, deg2, agg_t, a1, a2, g, b, wb, post_flag)


def _prescale_body(x_ref, deg_ref, g_ref, z_ref):
    n = x_ref.shape[1]
    dp = jax.nn.sigmoid(g_ref[0, 0])
    deg = jnp.maximum(deg_ref[0:1, :n] + deg_ref[1:2, :n], 1.0)
    f = jnp.exp((dp - 1.0) * jnp.log(deg))
    z_ref[...] = x_ref[...] * f


def _prep(x, edge_index):
    t, n = x.shape
    e = edge_index.shape[1]
    fast = (e % _UNIT == 0 and n % _XCH == 0
            and (e // _UNIT) // _NW >= 4 * _K)
    if fast:
        return x, edge_index.reshape(2, e // _UNIT, _UNIT)
    n_pad = _round_up(n, _XCH)
    e_pad = _round_up(max(e, 4 * _K * _NW * _UNIT), _NW * _UNIT)
    xin = jnp.pad(x, ((0, 0), (0, n_pad - n)))
    pad_idx = n_pad + (jnp.arange(e_pad - e, dtype=jnp.int32) % 32)
    edges = jnp.concatenate(
        [edge_index, jnp.broadcast_to(pad_idx, (2, e_pad - e))], axis=1
    ).reshape(2, e_pad // _UNIT, _UNIT)
    return xin, edges


def kernel(x, edge_index, alpha1, alpha2, gamma, bias, transpose, with_bias):
    t, n = x.shape
    xin, edges = _prep(x, edge_index)
    n_x = xin.shape[1]
    n_tbl = _round_up(n_x + 64, _NS * 3200)
    e_rows = edges.shape[1]

    scatter = _sc_scatter(n_tbl, e_rows, t, n_x)
    agg_t2, deg_flat = scatter(xin, edges)
    deg2 = deg_flat.reshape(_NC, n_tbl)
    wb = jnp.asarray(with_bias, jnp.int32).reshape(1, 1)

    def branch_plain(_):
        pf = jnp.ones((1, 1), jnp.int32)
        return _combine(x, deg2, agg_t2, alpha1, alpha2, gamma, bias, wb, pf)

    def branch_transpose(_):
        xs = pl.pallas_call(
            _prescale_body,
            out_shape=jax.ShapeDtypeStruct((t, n_x), jnp.float32),
        )(xin, deg2, gamma)
        agg_t2b, _ = scatter(xs, _prep(x, edge_index)[1])
        pf = jnp.zeros((1, 1), jnp.int32)
        return _combine(x, deg2, agg_t2b, alpha1, alpha2, gamma, bias, wb, pf)

    return lax.cond(jnp.asarray(transpose) == 0,
                    branch_plain, branch_transpose, operand=None)


# speedup vs baseline: 1.1888x; 1.0384x over previous
import functools

import jax
import jax.numpy as jnp
from jax import lax
from jax.experimental import pallas as pl
from jax.experimental.pallas import tpu as pltpu
from jax.experimental.pallas import tpu_sc as plsc

_NC = 2
_NS = 16
_NW = _NC * _NS
_UNIT = 512
_K = 4
_XCH = 400

_MESH = plsc.VectorSubcoreMesh(core_axis_name="c", subcore_axis_name="s")
_SC_PARAMS = pltpu.CompilerParams(use_tc_tiling_on_sc=False,
                                  needs_layout_passes=False)


def _round_up(a, b):
    return (a + b - 1) // b * b


@functools.lru_cache(maxsize=None)
def _sc_table(n_tbl, t, n_x):
    npt = n_tbl // _NS

    def body(xin, tbl, xbuf, tbuf, xsem):
        c = lax.axis_index("c")
        s = lax.axis_index("s")
        r0 = s * npt
        iota = lax.iota(jnp.int32, 16)

        for ch in range(npt // _XCH):
            c0 = r0 + ch * _XCH

            @pl.when(c0 + _XCH <= n_x)
            def _():
                pltpu.async_copy(xin.at[:, pl.ds(c0, _XCH)],
                                 xbuf.at[:, pl.ds(ch * _XCH, _XCH)], xsem)

        for ch in range(npt // _XCH):
            c0 = r0 + ch * _XCH

            @pl.when(c0 + _XCH <= n_x)
            def _():
                pltpu.make_async_copy(xin.at[:, pl.ds(c0, _XCH)],
                                      xbuf.at[:, pl.ds(ch * _XCH, _XCH)],
                                      xsem).wait()

        @pl.loop(0, npt // 16)
        def _tr(i):
            r16 = i * 16 + iota
            for tt in range(t):
                ft = jnp.full((16,), tt, jnp.int32)
                v = plsc.load_gather(xbuf, [ft, r16])
                plsc.store_scatter(tbuf, [r16, ft], v)

        pltpu.sync_copy(tbuf, tbl.at[c, pl.ds(r0, npt)])

    return pl.kernel(
        body,
        out_type=jax.ShapeDtypeStruct((_NC, n_tbl, t), jnp.float32),
        mesh=_MESH,
        scratch_types=[
            pltpu.VMEM((t, npt), jnp.float32),
            pltpu.VMEM((npt, t), jnp.float32),
            pltpu.SemaphoreType.DMA,
        ],
        compiler_params=_SC_PARAMS,
    )


@functools.lru_cache(maxsize=None)
def _sc_edges(n_tbl, e_rows, t):
    npt = n_tbl // _NS
    q, rem = divmod(e_rows, _NW)

    def body(tbl, ei, agg_out, deg_out,
             acc, deg_acc, src_v0, src_v1, dst_v0, dst_v1, rows_v0, rows_v1,
             xbuf, tbuf, zb1, ones_v,
             isem0, isem1, gsem0, gsem1, dsem0, dsem1, asem0, asem1):
        c = lax.axis_index("c")
        s = lax.axis_index("s")
        w = c * _NS + s
        r0 = s * npt
        srcs = (src_v0, src_v1)
        dsts = (dst_v0, dst_v1)
        rows = (rows_v0, rows_v1)
        isems = (isem0, isem1)
        gsems = (gsem0, gsem1)
        dsems = (dsem0, dsem1)
        asems = (asem0, asem1)
        iota = lax.iota(jnp.int32, 16)
        z16 = jnp.zeros((16,), jnp.float32)

        for i in range(_K):
            for j in range(_UNIT // 16):
                ones_v[i, pl.ds(j * 16, 16)] = jnp.ones((16,), jnp.float32)

        @pl.loop(0, npt // 16)
        def _z1(i):
            zb1[pl.ds(i * 16, 16)] = z16

        @pl.loop(0, npt // 16)
        def _z2(i):
            r16 = i * 16 + iota
            for tt in range(t):
                plsc.store_scatter(tbuf, [r16, jnp.full((16,), tt, jnp.int32)],
                                   z16)

        pltpu.sync_copy(tbuf, acc.at[pl.ds(r0, npt)])
        pltpu.sync_copy(zb1, deg_acc.at[pl.ds(r0, npt)])
        plsc.subcore_barrier()

        base = w * q + jnp.minimum(w, rem)
        nrows = q + jnp.where(w < rem, 1, 0)
        nfull = (nrows // (2 * _K)) * 2
        tblc = tbl.at[c]

        def fire_idx(b, p):
            rb = base + b * _K
            pltpu.async_copy(ei.at[0, pl.ds(rb, _K)], srcs[p], isems[p])
            pltpu.async_copy(ei.at[1, pl.ds(rb, _K)], dsts[p], isems[p])

        def wait_idx(b, p):
            rb = base + b * _K
            pltpu.make_async_copy(ei.at[0, pl.ds(rb, _K)], srcs[p],
                                  isems[p]).wait()
            pltpu.make_async_copy(ei.at[1, pl.ds(rb, _K)], dsts[p],
                                  isems[p]).wait()

        def drain_scatters(p):
            for j in range(_K):
                pltpu.make_async_copy(ones_v.at[j], deg_acc.at[srcs[p].at[j]],
                                      dsems[p]).wait()
            for j in range(_K):
                pltpu.make_async_copy(rows[p].at[j], acc.at[dsts[p].at[j]],
                                      asems[p]).wait()

        def block(b, p, drain_prev):
            wait_idx(b, p)
            gds = []
            for j in range(_K):
                gds.append(pltpu.async_copy(tblc.at[srcs[p].at[j]],
                                            rows[p].at[j], gsems[p]))
            for j in range(_K):
                pltpu.async_copy(ones_v.at[j], deg_acc.at[srcs[p].at[j]],
                                 dsems[p], add=True)
            if drain_prev:
                drain_scatters(1 - p)

            @pl.when(b + 1 < nfull)
            def _():
                fire_idx(b + 1, 1 - p)

            for d in gds:
                d.wait()
            for j in range(_K):
                pltpu.async_copy(rows[p].at[j], acc.at[dsts[p].at[j]],
                                 asems[p], add=True)

        fire_idx(0, 0)
        block(0, 0, drain_prev=False)
        block(1, 1, drain_prev=True)

        @pl.loop(1, nfull // 2)
        def _outer(gg):
            block(gg * 2, 0, drain_prev=True)
            block(gg * 2 + 1, 1, drain_prev=True)

        drain_scatters(1)

        @pl.loop(nfull * _K, nrows)
        def _tail(r):
            pltpu.sync_copy(ei.at[0, pl.ds(base + r, 1)],
                            srcs[0].at[pl.ds(0, 1)])
            pltpu.sync_copy(ei.at[1, pl.ds(base + r, 1)],
                            dsts[0].at[pl.ds(0, 1)])
            pltpu.sync_copy(tblc.at[srcs[0].at[0]], rows[0].at[0])
            pltpu.sync_copy(ones_v.at[0], deg_acc.at[srcs[0].at[0]], add=True)
            pltpu.sync_copy(rows[0].at[0], acc.at[dsts[0].at[0]], add=True)

        plsc.subcore_barrier()

        pltpu.sync_copy(deg_acc.at[pl.ds(r0, npt)],
                        deg_out.at[pl.ds(c * n_tbl + r0, npt)])
        pltpu.sync_copy(acc.at[pl.ds(r0, npt)], tbuf)

        @pl.loop(0, npt // 16)
        def _tro(i):
            r16 = i * 16 + iota
            for tt in range(t):
                v = plsc.load_gather(tbuf, [r16, jnp.full((16,), tt,
                                                          jnp.int32)])
                xbuf[tt, pl.ds(i * 16, 16)] = v

        pltpu.sync_copy(xbuf, agg_out.at[c, :, pl.ds(r0, npt)])

    return pl.kernel(
        body,
        out_type=(
            jax.ShapeDtypeStruct((_NC, t, n_tbl), jnp.float32),
            jax.ShapeDtypeStruct((_NC * n_tbl,), jnp.float32),
        ),
        mesh=_MESH,
        scratch_types=(
            [pltpu.VMEM_SHARED((n_tbl, t), jnp.float32),
             pltpu.VMEM_SHARED((n_tbl,), jnp.float32)]
            + [pltpu.VMEM((_K, _UNIT), jnp.int32)] * 4
            + [pltpu.VMEM((_K, _UNIT, t), jnp.float32)] * 2
            + [pltpu.VMEM((t, npt), jnp.float32),
               pltpu.VMEM((npt, t), jnp.float32),
               pltpu.VMEM((npt,), jnp.float32),
               pltpu.VMEM((_K, _UNIT), jnp.float32)]
            + [pltpu.SemaphoreType.DMA] * 8
        ),
        compiler_params=_SC_PARAMS,
    )


def _combine_body(x_ref, deg_ref, agg_ref, a1_ref, a2_ref, g_ref, b_ref,
                  wb_ref, pf_ref, out_ref):
    n = x_ref.shape[1]
    deg = jnp.maximum(deg_ref[0:1, :n] + deg_ref[1:2, :n], 1.0)
    ld = jnp.log(deg)
    dp = jax.nn.sigmoid(g_ref[0, 0])
    sw = jnp.exp(a1_ref[0, 0])
    nw = sw * jnp.tanh(a2_ref[0, 0])
    agg = agg_ref[0, :, :n] + agg_ref[1, :, :n]
    wr = x_ref[...] * jnp.exp(dp * ld)
    post = jnp.where(pf_ref[0, 0] != 0,
                     jnp.exp((dp - 1.0) * ld), jnp.ones_like(ld))
    outv = sw * wr + nw * agg * post
    out_ref[...] = outv + jnp.where(wb_ref[0, 0] != 0, b_ref[0, 0], 0.0)


def _combine(x, deg2, agg_t, a1, a2, g, b, wb, post_flag):
    t, n = x.shape
    return pl.pallas_call(
        _combine_body,
        out_shape=jax.ShapeDtypeStruct((t, n), jnp.float32),
    )(x, deg2, agg_t, a1, a2, g, b, wb, post_flag)


def _prescale_body(x_ref, deg_ref, g_ref, z_ref):
    n = x_ref.shape[1]
    dp = jax.nn.sigmoid(g_ref[0, 0])
    deg = jnp.maximum(deg_ref[0:1, :n] + deg_ref[1:2, :n], 1.0)
    f = jnp.exp((dp - 1.0) * jnp.log(deg))
    z_ref[...] = x_ref[...] * f


def _prep(x, edge_index):
    t, n = x.shape
    e = edge_index.shape[1]
    fast = (e % _UNIT == 0 and n % _XCH == 0
            and (e // _UNIT) // _NW >= 4 * _K)
    if fast:
        return x, edge_index.reshape(2, e // _UNIT, _UNIT)
    n_pad = _round_up(n, _XCH)
    e_pad = _round_up(max(e, 4 * _K * _NW * _UNIT), _NW * _UNIT)
    xin = jnp.pad(x, ((0, 0), (0, n_pad - n)))
    pad_idx = n_pad + (jnp.arange(e_pad - e, dtype=jnp.int32) % 32)
    edges = jnp.concatenate(
        [edge_index, jnp.broadcast_to(pad_idx, (2, e_pad - e))], axis=1
    ).reshape(2, e_pad // _UNIT, _UNIT)
    return xin, edges


def kernel(x, edge_index, alpha1, alpha2, gamma, bias, transpose, with_bias):
    t, n = x.shape
    xin, edges = _prep(x, edge_index)
    n_x = xin.shape[1]
    n_tbl = _round_up(n_x + 64, _NS * 3200)
    e_rows = edges.shape[1]

    tbl = _sc_table(n_tbl, t, n_x)(xin)
    agg_t2, deg_flat = _sc_edges(n_tbl, e_rows, t)(tbl, edges)
    deg2 = deg_flat.reshape(_NC, n_tbl)
    wb = jnp.asarray(with_bias, jnp.int32).reshape(1, 1)

    def branch_plain(_):
        pf = jnp.ones((1, 1), jnp.int32)
        return _combine(x, deg2, agg_t2, alpha1, alpha2, gamma, bias, wb, pf)

    def branch_transpose(_):
        xs = pl.pallas_call(
            _prescale_body,
            out_shape=jax.ShapeDtypeStruct((t, n_x), jnp.float32),
        )(xin, deg2, gamma)
        tbl_s = _sc_table(n_tbl, t, n_x)(xs)
        agg_t2b, _ = _sc_edges(n_tbl, e_rows, t)(
            tbl_s, _prep(x, edge_index)[1])
        pf = jnp.zeros((1, 1), jnp.int32)
        return _combine(x, deg2, agg_t2b, alpha1, alpha2, gamma, bias, wb, pf)

    return lax.cond(jnp.asarray(transpose) == 0,
                    branch_plain, branch_transpose, operand=None)


# speedup vs baseline: 1.1972x; 1.0071x over previous
import functools

import jax
import jax.numpy as jnp
from jax import lax
from jax.experimental import pallas as pl
from jax.experimental.pallas import tpu as pltpu
from jax.experimental.pallas import tpu_sc as plsc

_NC = 2
_NS = 16
_NW = _NC * _NS
_UNIT = 512
_K = 4
_XCH = 400

_MESH = plsc.VectorSubcoreMesh(core_axis_name="c", subcore_axis_name="s")
_SC_PARAMS = pltpu.CompilerParams(use_tc_tiling_on_sc=False,
                                  needs_layout_passes=False)


def _round_up(a, b):
    return (a + b - 1) // b * b


@functools.lru_cache(maxsize=None)
def _sc_table(n_tbl, t, n_x):
    npt = n_tbl // _NS

    def body(xin, tbl, xbuf, tbuf, xsem):
        c = lax.axis_index("c")
        s = lax.axis_index("s")
        r0 = s * npt
        iota = lax.iota(jnp.int32, 16)

        for ch in range(npt // _XCH):
            c0 = r0 + ch * _XCH

            @pl.when(c0 + _XCH <= n_x)
            def _():
                pltpu.async_copy(xin.at[:, pl.ds(c0, _XCH)],
                                 xbuf.at[:, pl.ds(ch * _XCH, _XCH)], xsem)

        for ch in range(npt // _XCH):
            c0 = r0 + ch * _XCH

            @pl.when(c0 + _XCH <= n_x)
            def _():
                pltpu.make_async_copy(xin.at[:, pl.ds(c0, _XCH)],
                                      xbuf.at[:, pl.ds(ch * _XCH, _XCH)],
                                      xsem).wait()

        @pl.loop(0, npt // 16)
        def _tr(i):
            r16 = i * 16 + iota
            for tt in range(t):
                ft = jnp.full((16,), tt, jnp.int32)
                v = plsc.load_gather(xbuf, [ft, r16])
                plsc.store_scatter(tbuf, [r16, ft], v)

        pltpu.sync_copy(tbuf, tbl.at[c, pl.ds(r0, npt)])

    return pl.kernel(
        body,
        out_type=jax.ShapeDtypeStruct((_NC, n_tbl, t), jnp.float32),
        mesh=_MESH,
        scratch_types=[
            pltpu.VMEM((t, npt), jnp.float32),
            pltpu.VMEM((npt, t), jnp.float32),
            pltpu.SemaphoreType.DMA,
        ],
        compiler_params=_SC_PARAMS,
    )


@functools.lru_cache(maxsize=None)
def _sc_edges(n_tbl, e_rows, t):
    npt = n_tbl // _NS
    q, rem = divmod(e_rows, _NW)

    def body(tbl, ei, agg_out, deg_out,
             acc, deg_acc, src_v0, src_v1, dst_v0, dst_v1, rows_v0, rows_v1,
             xbuf, tbuf, zb1, ones_v,
             isem0, isem1, gsem0, gsem1, dsem0, dsem1, asem0, asem1):
        c = lax.axis_index("c")
        s = lax.axis_index("s")
        w = c * _NS + s
        r0 = s * npt
        srcs = (src_v0, src_v1)
        dsts = (dst_v0, dst_v1)
        rows = (rows_v0, rows_v1)
        isems = (isem0, isem1)
        gsems = (gsem0, gsem1)
        dsems = (dsem0, dsem1)
        asems = (asem0, asem1)
        iota = lax.iota(jnp.int32, 16)
        z16 = jnp.zeros((16,), jnp.float32)

        for i in range(_K):
            for j in range(_UNIT // 16):
                ones_v[i, pl.ds(j * 16, 16)] = jnp.ones((16,), jnp.float32)

        @pl.loop(0, npt // 16)
        def _z1(i):
            zb1[pl.ds(i * 16, 16)] = z16

        @pl.loop(0, npt // 16)
        def _z2(i):
            r16 = i * 16 + iota
            for tt in range(t):
                plsc.store_scatter(tbuf, [r16, jnp.full((16,), tt, jnp.int32)],
                                   z16)

        pltpu.sync_copy(tbuf, acc.at[pl.ds(r0, npt)])
        pltpu.sync_copy(zb1, deg_acc.at[pl.ds(r0, npt)])
        plsc.subcore_barrier()

        base = w * q + jnp.minimum(w, rem)
        nrows = q + jnp.where(w < rem, 1, 0)
        nfull = (nrows // (2 * _K)) * 2
        tblc = tbl.at[c]

        def fire_idx(b, p):
            rb = base + b * _K
            pltpu.async_copy(ei.at[0, pl.ds(rb, _K)], srcs[p], isems[p])
            pltpu.async_copy(ei.at[1, pl.ds(rb, _K)], dsts[p], isems[p])

        def wait_idx(b, p):
            rb = base + b * _K
            pltpu.make_async_copy(ei.at[0, pl.ds(rb, _K)], srcs[p],
                                  isems[p]).wait()
            pltpu.make_async_copy(ei.at[1, pl.ds(rb, _K)], dsts[p],
                                  isems[p]).wait()

        def drain_scatters(p):
            for j in range(_K):
                pltpu.make_async_copy(ones_v.at[j], deg_acc.at[srcs[p].at[j]],
                                      dsems[p]).wait()
            for j in range(_K):
                pltpu.make_async_copy(rows[p].at[j], acc.at[dsts[p].at[j]],
                                      asems[p]).wait()

        def block(b, p, drain_prev):
            wait_idx(b, p)
            gds = []
            for j in range(_K):
                gds.append(pltpu.async_copy(tblc.at[srcs[p].at[j]],
                                            rows[p].at[j], gsems[p]))
            for j in range(_K):
                pltpu.async_copy(ones_v.at[j], deg_acc.at[srcs[p].at[j]],
                                 dsems[p], add=True)
            if drain_prev:
                drain_scatters(1 - p)

            @pl.when(b + 1 < nfull)
            def _():
                fire_idx(b + 1, 1 - p)

            for d in gds:
                d.wait()
            for j in range(_K):
                pltpu.async_copy(rows[p].at[j], acc.at[dsts[p].at[j]],
                                 asems[p], add=True)

        fire_idx(0, 0)
        block(0, 0, drain_prev=False)
        block(1, 1, drain_prev=True)

        @pl.loop(1, nfull // 2)
        def _outer(gg):
            block(gg * 2, 0, drain_prev=True)
            block(gg * 2 + 1, 1, drain_prev=True)

        drain_scatters(1)

        @pl.loop(nfull * _K, nrows)
        def _tail(r):
            pltpu.sync_copy(ei.at[0, pl.ds(base + r, 1)],
                            srcs[0].at[pl.ds(0, 1)])
            pltpu.sync_copy(ei.at[1, pl.ds(base + r, 1)],
                            dsts[0].at[pl.ds(0, 1)])
            pltpu.sync_copy(tblc.at[srcs[0].at[0]], rows[0].at[0])
            pltpu.sync_copy(ones_v.at[0], deg_acc.at[srcs[0].at[0]], add=True)
            pltpu.sync_copy(rows[0].at[0], acc.at[dsts[0].at[0]], add=True)

        plsc.subcore_barrier()

        pltpu.sync_copy(deg_acc.at[pl.ds(r0, npt)],
                        deg_out.at[pl.ds(c * n_tbl + r0, npt)])
        pltpu.sync_copy(acc.at[pl.ds(r0, npt)], tbuf)

        @pl.loop(0, npt // 16)
        def _tro(i):
            r16 = i * 16 + iota
            for tt in range(t):
                v = plsc.load_gather(tbuf, [r16, jnp.full((16,), tt,
                                                          jnp.int32)])
                xbuf[tt, pl.ds(i * 16, 16)] = v

        pltpu.sync_copy(xbuf, agg_out.at[c, :, pl.ds(r0, npt)])

    return pl.kernel(
        body,
        out_type=(
            jax.ShapeDtypeStruct((_NC, t, n_tbl), jnp.float32),
            jax.ShapeDtypeStruct((_NC * n_tbl,), jnp.float32),
        ),
        mesh=_MESH,
        scratch_types=(
            [pltpu.VMEM_SHARED((n_tbl, t), jnp.float32),
             pltpu.VMEM_SHARED((n_tbl,), jnp.float32)]
            + [pltpu.VMEM((_K, _UNIT), jnp.int32)] * 4
            + [pltpu.VMEM((_K, _UNIT, t), jnp.float32)] * 2
            + [pltpu.VMEM((t, npt), jnp.float32),
               pltpu.VMEM((npt, t), jnp.float32),
               pltpu.VMEM((npt,), jnp.float32),
               pltpu.VMEM((_K, _UNIT), jnp.float32)]
            + [pltpu.SemaphoreType.DMA] * 8
        ),
        compiler_params=_SC_PARAMS,
    )


def _combine_body(x_ref, deg_ref, agg_ref, a1_ref, a2_ref, g_ref, b_ref,
                  wb_ref, pf_ref, out_ref):
    n = x_ref.shape[1]
    n_tbl = deg_ref.shape[1] // 2
    deg = jnp.maximum(deg_ref[0:1, :n] + deg_ref[0:1, n_tbl:n_tbl + n],
                      1.0)
    ld = jnp.log(deg)
    dp = jax.nn.sigmoid(g_ref[0, 0])
    sw = jnp.exp(a1_ref[0, 0])
    nw = sw * jnp.tanh(a2_ref[0, 0])
    agg = agg_ref[0, :, :n] + agg_ref[1, :, :n]
    wr = x_ref[...] * jnp.exp(dp * ld)
    post = jnp.where(pf_ref[0, 0] != 0,
                     jnp.exp((dp - 1.0) * ld), jnp.ones_like(ld))
    outv = sw * wr + nw * agg * post
    out_ref[...] = outv + jnp.where(wb_ref[0, 0] != 0, b_ref[0, 0], 0.0)


def _combine(x, deg2, agg_t, a1, a2, g, b, wb, post_flag):
    t, n = x.shape
    return pl.pallas_call(
        _combine_body,
        out_shape=jax.ShapeDtypeStruct((t, n), jnp.float32),
    )(x, deg2, agg_t, a1, a2, g, b, wb, post_flag)


def _prescale_body(x_ref, deg_ref, g_ref, z_ref):
    n = x_ref.shape[1]
    n_tbl = deg_ref.shape[1] // 2
    deg = jnp.maximum(deg_ref[0:1, :n] + deg_ref[0:1, n_tbl:n_tbl + n],
                      1.0)
    dp = jax.nn.sigmoid(g_ref[0, 0])
    f = jnp.exp((dp - 1.0) * jnp.log(deg))
    z_ref[...] = x_ref[...] * f


def _prep(x, edge_index):
    t, n = x.shape
    e = edge_index.shape[1]
    fast = (e % _UNIT == 0 and n % _XCH == 0
            and (e // _UNIT) // _NW >= 4 * _K)
    if fast:
        return x, edge_index.reshape(2, e // _UNIT, _UNIT)
    n_pad = _round_up(n, _XCH)
    e_pad = _round_up(max(e, 4 * _K * _NW * _UNIT), _NW * _UNIT)
    xin = jnp.pad(x, ((0, 0), (0, n_pad - n)))
    pad_idx = n_pad + (jnp.arange(e_pad - e, dtype=jnp.int32) % 32)
    edges = jnp.concatenate(
        [edge_index, jnp.broadcast_to(pad_idx, (2, e_pad - e))], axis=1
    ).reshape(2, e_pad // _UNIT, _UNIT)
    return xin, edges


def kernel(x, edge_index, alpha1, alpha2, gamma, bias, transpose, with_bias):
    t, n = x.shape
    xin, edges = _prep(x, edge_index)
    n_x = xin.shape[1]
    n_tbl = _round_up(n_x + 64, _NS * 3200)
    e_rows = edges.shape[1]

    tbl = _sc_table(n_tbl, t, n_x)(xin)
    agg_t2, deg_flat = _sc_edges(n_tbl, e_rows, t)(tbl, edges)
    deg2 = deg_flat.reshape(1, _NC * n_tbl)
    wb = jnp.asarray(with_bias, jnp.int32).reshape(1, 1)

    def branch_plain(_):
        pf = jnp.ones((1, 1), jnp.int32)
        return _combine(x, deg2, agg_t2, alpha1, alpha2, gamma, bias, wb, pf)

    def branch_transpose(_):
        xs = pl.pallas_call(
            _prescale_body,
            out_shape=jax.ShapeDtypeStruct((t, n_x), jnp.float32),
        )(xin, deg2, gamma)
        tbl_s = _sc_table(n_tbl, t, n_x)(xs)
        agg_t2b, _ = _sc_edges(n_tbl, e_rows, t)(
            tbl_s, _prep(x, edge_index)[1])
        pf = jnp.zeros((1, 1), jnp.int32)
        return _combine(x, deg2, agg_t2b, alpha1, alpha2, gamma, bias, wb, pf)

    return lax.cond(jnp.asarray(transpose) == 0,
                    branch_plain, branch_transpose, operand=None)
